# initial kernel scaffold (unmeasured)
import jax
import jax.numpy as jnp
from jax import lax
from jax.experimental import pallas as pl
from jax.experimental.pallas import tpu as pltpu

M_PER = 2048
D_DIM = 2048
F_DIM = 8192
Q = 512
TF = 512
N_TF = F_DIM // TF


def kernel(x, dy):
    def body(x_hbm, dy_hbm, out_hbm,
             xstage, xall, dystage, e_buf, d_buf, r1_buf, r2_buf, out_stage,
             local_sem, out_sem, y_send_sem, y_recv_sem, z_send_sem, z_recv_sem):
        my_x = lax.axis_index("x")
        my_y = lax.axis_index("y")
        my_z = lax.axis_index("z")

        barrier_sem = pltpu.get_barrier_semaphore()
        pl.semaphore_signal(barrier_sem, inc=1,
                            device_id=(my_x, 1 - my_y, my_z),
                            device_id_type=pl.DeviceIdType.MESH)
        pl.semaphore_signal(barrier_sem, inc=1,
                            device_id=(my_x, my_y, 1 - my_z),
                            device_id_type=pl.DeviceIdType.MESH)
        pl.semaphore_wait(barrier_sem, 2)

        qe_col = (1 - my_y) * 1024 + my_z * Q
        qd_col = my_y * 1024
        for i, coloff in enumerate([qe_col, qd_col, qd_col + Q]):
            cp = pltpu.make_async_copy(
                x_hbm.at[:, pl.ds(coloff, Q)], xstage, local_sem)
            cp.start()
            cp.wait()
            xall[:, i * Q:(i + 1) * Q] = xstage[:, :].astype(jnp.bfloat16)

        for t in range(N_TF):
            cp = pltpu.make_async_copy(
                dy_hbm.at[:, pl.ds(t * TF, TF)], dystage, local_sem)
            cp.start()
            cp.wait()
            dyt = dystage[:, :].astype(jnp.bfloat16)
            res = lax.dot_general(
                xall[:, :], dyt,
                dimension_numbers=(((0,), (0,)), ((), ())),
                preferred_element_type=jnp.float32,
            )
            e_buf[:, t * TF:(t + 1) * TF] = res[0:Q, :].astype(jnp.bfloat16)
            d_buf[:, t * TF:(t + 1) * TF] = res[Q:, :].astype(jnp.bfloat16)

        rdma_y = pltpu.make_async_remote_copy(
            src_ref=e_buf, dst_ref=r1_buf,
            send_sem=y_send_sem, recv_sem=y_recv_sem,
            device_id=(my_x, 1 - my_y, my_z),
            device_id_type=pl.DeviceIdType.MESH,
        )
        rdma_y.start()
        rdma_y.wait()

        rdma_z = pltpu.make_async_remote_copy(
            src_ref=r1_buf, dst_ref=r2_buf,
            send_sem=z_send_sem, recv_sem=z_recv_sem,
            device_id=(my_x, my_y, 1 - my_z),
            device_id_type=pl.DeviceIdType.MESH,
        )
        rdma_z.start()
        rdma_z.wait()

        z_is_0 = (my_z == 0)
        for t in range(N_TF):
            sl = pl.ds(t * TF, TF)
            d = d_buf[:, sl].astype(jnp.float32)
            r1 = r1_buf[:, sl].astype(jnp.float32)
            r2 = r2_buf[:, sl].astype(jnp.float32)
            top = d[0:Q, :] + jnp.where(z_is_0, r1, r2)
            bot = d[Q:, :] + jnp.where(z_is_0, r2, r1)
            out_stage[0:Q, :] = top
            out_stage[Q:, :] = bot
            cp = pltpu.make_async_copy(
                out_stage, out_hbm.at[:, sl], out_sem)
            cp.start()
            cp.wait()

    return pl.pallas_call(
        body,
        out_shape=jax.ShapeDtypeStruct((D_DIM // 2, F_DIM), jnp.float32),
        in_specs=[
            pl.BlockSpec(memory_space=pltpu.ANY),
            pl.BlockSpec(memory_space=pltpu.ANY),
        ],
        out_specs=pl.BlockSpec(memory_space=pltpu.ANY),
        scratch_shapes=[
            pltpu.VMEM((M_PER, Q), jnp.float32),
            pltpu.VMEM((M_PER, 3 * Q), jnp.bfloat16),
            pltpu.VMEM((M_PER, TF), jnp.float32),
            pltpu.VMEM((Q, F_DIM), jnp.bfloat16),
            pltpu.VMEM((2 * Q, F_DIM), jnp.bfloat16),
            pltpu.VMEM((Q, F_DIM), jnp.bfloat16),
            pltpu.VMEM((Q, F_DIM), jnp.bfloat16),
            pltpu.VMEM((2 * Q, TF), jnp.float32),
            pltpu.SemaphoreType.DMA,
            pltpu.SemaphoreType.DMA,
            pltpu.SemaphoreType.DMA,
            pltpu.SemaphoreType.DMA,
            pltpu.SemaphoreType.DMA,
            pltpu.SemaphoreType.DMA,
        ],
        compiler_params=pltpu.CompilerParams(collective_id=0),
    )(x, dy)


# baseline (device time: 420206 ns/iter reference)
import jax
import jax.numpy as jnp
from jax import lax
from jax.experimental import pallas as pl
from jax.experimental.pallas import tpu as pltpu

M_PER = 2048
D_DIM = 2048
F_DIM = 8192
Q = 512
TF = 256
XC = 256
N_TF = F_DIM // TF


def kernel(x, dy):
    def body(x_hbm, dy_hbm, out_hbm,
             xstage, xall, dystage, e_buf, d_buf, r1_buf, r2_buf, out_stage,
             local_sem, out_sem, y_send_sem, y_recv_sem, z_send_sem, z_recv_sem):
        my_x = lax.axis_index("x")
        my_y = lax.axis_index("y")
        my_z = lax.axis_index("z")

        barrier_sem = pltpu.get_barrier_semaphore()
        pl.semaphore_signal(barrier_sem, inc=1,
                            device_id=(my_x, 1 - my_y, my_z),
                            device_id_type=pl.DeviceIdType.MESH)
        pl.semaphore_signal(barrier_sem, inc=1,
                            device_id=(my_x, my_y, 1 - my_z),
                            device_id_type=pl.DeviceIdType.MESH)
        pl.semaphore_wait(barrier_sem, 2)

        qe_col = (1 - my_y) * 1024 + my_z * Q
        qd_col = my_y * 1024
        for i, coloff in enumerate([qe_col, qd_col, qd_col + Q]):
            for c in range(Q // XC):
                cp = pltpu.make_async_copy(
                    x_hbm.at[:, pl.ds(coloff + c * XC, XC)], xstage, local_sem)
                cp.start()
                cp.wait()
                xall[:, i * Q + c * XC:i * Q + (c + 1) * XC] = (
                    xstage[:, :].astype(jnp.bfloat16))

        for t in range(N_TF):
            cp = pltpu.make_async_copy(
                dy_hbm.at[:, pl.ds(t * TF, TF)], dystage, local_sem)
            cp.start()
            cp.wait()
            dyt = dystage[:, :].astype(jnp.bfloat16)
            res = lax.dot_general(
                xall[:, :], dyt,
                dimension_numbers=(((0,), (0,)), ((), ())),
                preferred_element_type=jnp.float32,
            )
            e_buf[:, t * TF:(t + 1) * TF] = res[0:Q, :].astype(jnp.bfloat16)
            d_buf[:, t * TF:(t + 1) * TF] = res[Q:, :].astype(jnp.bfloat16)

        rdma_y = pltpu.make_async_remote_copy(
            src_ref=e_buf, dst_ref=r1_buf,
            send_sem=y_send_sem, recv_sem=y_recv_sem,
            device_id=(my_x, 1 - my_y, my_z),
            device_id_type=pl.DeviceIdType.MESH,
        )
        rdma_y.start()
        rdma_y.wait()

        rdma_z = pltpu.make_async_remote_copy(
            src_ref=r1_buf, dst_ref=r2_buf,
            send_sem=z_send_sem, recv_sem=z_recv_sem,
            device_id=(my_x, my_y, 1 - my_z),
            device_id_type=pl.DeviceIdType.MESH,
        )
        rdma_z.start()
        rdma_z.wait()

        z_is_0 = (my_z == 0)
        for t in range(N_TF):
            sl = pl.ds(t * TF, TF)
            d = d_buf[:, sl].astype(jnp.float32)
            r1 = r1_buf[:, sl].astype(jnp.float32)
            r2 = r2_buf[:, sl].astype(jnp.float32)
            top = d[0:Q, :] + jnp.where(z_is_0, r1, r2)
            bot = d[Q:, :] + jnp.where(z_is_0, r2, r1)
            out_stage[0:Q, :] = top
            out_stage[Q:, :] = bot
            cp = pltpu.make_async_copy(
                out_stage, out_hbm.at[:, sl], out_sem)
            cp.start()
            cp.wait()

    return pl.pallas_call(
        body,
        out_shape=jax.ShapeDtypeStruct((D_DIM // 2, F_DIM), jnp.float32),
        in_specs=[
            pl.BlockSpec(memory_space=pl.ANY),
            pl.BlockSpec(memory_space=pl.ANY),
        ],
        out_specs=pl.BlockSpec(memory_space=pl.ANY),
        scratch_shapes=[
            pltpu.VMEM((M_PER, XC), jnp.float32),
            pltpu.VMEM((M_PER, 3 * Q), jnp.bfloat16),
            pltpu.VMEM((M_PER, TF), jnp.float32),
            pltpu.VMEM((Q, F_DIM), jnp.bfloat16),
            pltpu.VMEM((2 * Q, F_DIM), jnp.bfloat16),
            pltpu.VMEM((Q, F_DIM), jnp.bfloat16),
            pltpu.VMEM((Q, F_DIM), jnp.bfloat16),
            pltpu.VMEM((2 * Q, TF), jnp.float32),
            pltpu.SemaphoreType.DMA,
            pltpu.SemaphoreType.DMA,
            pltpu.SemaphoreType.DMA,
            pltpu.SemaphoreType.DMA,
            pltpu.SemaphoreType.DMA,
            pltpu.SemaphoreType.DMA,
        ],
        compiler_params=pltpu.CompilerParams(
            collective_id=0,
            vmem_limit_bytes=64 * 1024 * 1024,
        ),
    )(x, dy)


# device time: 191159 ns/iter; 2.1982x vs baseline; 2.1982x over previous
import jax
import jax.numpy as jnp
from jax import lax
from jax.experimental import pallas as pl
from jax.experimental.pallas import tpu as pltpu

M_PER = 2048
D_DIM = 2048
F_DIM = 8192
Q = 512
TF = 256
N_TF = F_DIM // TF
XC = 256
NC = 8
CF = F_DIM // NC
TPC = CF // TF


def kernel(x, dy):
    def body(x_hbm, dy_hbm, out_hbm,
             xstage, xall, dystage, e_buf, d_buf, r1_buf, r2_buf, out_stage,
             local_sems, out_sem, y_send_sems, y_recv_sems,
             z_send_sems, z_recv_sems):
        my_x = lax.axis_index("x")
        my_y = lax.axis_index("y")
        my_z = lax.axis_index("z")
        y_nbr = (my_x, 1 - my_y, my_z)
        z_nbr = (my_x, my_y, 1 - my_z)

        barrier_sem = pltpu.get_barrier_semaphore()
        for nbr in (y_nbr, z_nbr):
            pl.semaphore_signal(barrier_sem, inc=1, device_id=nbr,
                                device_id_type=pl.DeviceIdType.MESH)
        pl.semaphore_wait(barrier_sem, 2)

        def y_rdma(i):
            return pltpu.make_async_remote_copy(
                src_ref=e_buf.at[i], dst_ref=r1_buf.at[i],
                send_sem=y_send_sems.at[i], recv_sem=y_recv_sems.at[i],
                device_id=y_nbr, device_id_type=pl.DeviceIdType.MESH)

        def z_rdma(i):
            return pltpu.make_async_remote_copy(
                src_ref=r1_buf.at[i], dst_ref=r2_buf.at[i],
                send_sem=z_send_sems.at[i], recv_sem=z_recv_sems.at[i],
                device_id=z_nbr, device_id_type=pl.DeviceIdType.MESH)

        qe_col = (1 - my_y) * 1024 + my_z * Q
        qd_col = my_y * 1024
        xoffs = []
        for i, coloff in enumerate([qe_col, qd_col, qd_col + Q]):
            for c in range(Q // XC):
                xoffs.append((i * Q + c * XC, coloff + c * XC))
        cps = [None, None]
        for j, (dst_off, src_off) in enumerate(xoffs):
            slot = j % 2
            cps[slot] = pltpu.make_async_copy(
                x_hbm.at[:, pl.ds(src_off, XC)], xstage.at[slot],
                local_sems.at[slot])
            cps[slot].start()
            if j > 0:
                prev_dst = xoffs[j - 1][0]
                cps[(j - 1) % 2].wait()
                xall[:, prev_dst:prev_dst + XC] = (
                    xstage[(j - 1) % 2].astype(jnp.bfloat16))
        cps[(len(xoffs) - 1) % 2].wait()
        last_dst = xoffs[-1][0]
        xall[:, last_dst:last_dst + XC] = (
            xstage[(len(xoffs) - 1) % 2].astype(jnp.bfloat16))

        def dy_cp(t, slot):
            return pltpu.make_async_copy(
                dy_hbm.at[:, pl.ds(t * TF, TF)], dystage.at[slot],
                local_sems.at[slot])

        dy_cp(0, 0).start()

        def compute_step(t, carry):
            slot = t % 2

            @pl.when(t + 1 < N_TF)
            def _():
                dy_cp(t + 1, 1 - slot).start()

            dy_cp(t, slot).wait()
            dyt = dystage[slot].astype(jnp.bfloat16)
            res = lax.dot_general(
                xall[:, :], dyt,
                dimension_numbers=(((0,), (0,)), ((), ())),
                preferred_element_type=jnp.float32,
            )
            ci, cc = t // TPC, (t % TPC) * TF
            e_buf[ci, :, pl.ds(cc, TF)] = res[0:Q, :].astype(jnp.bfloat16)
            d_buf[:, pl.ds(t * TF, TF)] = res[Q:, :].astype(jnp.bfloat16)

            @pl.when(t % TPC == TPC - 1)
            def _():
                y_rdma(ci).start()

            return carry

        lax.fori_loop(0, N_TF, compute_step, 0)

        z_is_0 = (my_z == 0)

        def emit_out(i):
            d = d_buf[:, pl.ds(i * CF, CF)].astype(jnp.float32)
            r1 = r1_buf[i].astype(jnp.float32)
            r2 = r2_buf[i].astype(jnp.float32)
            out_stage[0:Q, :] = d[0:Q, :] + jnp.where(z_is_0, r1, r2)
            out_stage[Q:, :] = d[Q:, :] + jnp.where(z_is_0, r2, r1)
            pltpu.make_async_copy(
                out_stage, out_hbm.at[:, pl.ds(i * CF, CF)], out_sem).start()

        def out_wait(i):
            pltpu.make_async_copy(
                out_stage, out_hbm.at[:, pl.ds(i * CF, CF)], out_sem).wait()

        def pipe_step(i, carry):
            y_rdma(i).wait_recv()
            z_rdma(i).start()

            @pl.when(i > 0)
            def _():
                z_rdma(i - 1).wait_recv()

                @pl.when(i > 1)
                def _():
                    out_wait(i - 2)

                emit_out(i - 1)

            return carry

        lax.fori_loop(0, NC, pipe_step, 0)
        z_rdma(NC - 1).wait_recv()
        out_wait(NC - 2)
        emit_out(NC - 1)
        out_wait(NC - 1)

        def drain_step(i, carry):
            y_rdma(i).wait_send()
            z_rdma(i).wait_send()
            return carry

        lax.fori_loop(0, NC, drain_step, 0)

    return pl.pallas_call(
        body,
        out_shape=jax.ShapeDtypeStruct((D_DIM // 2, F_DIM), jnp.float32),
        in_specs=[
            pl.BlockSpec(memory_space=pl.ANY),
            pl.BlockSpec(memory_space=pl.ANY),
        ],
        out_specs=pl.BlockSpec(memory_space=pl.ANY),
        scratch_shapes=[
            pltpu.VMEM((2, M_PER, XC), jnp.float32),
            pltpu.VMEM((M_PER, 3 * Q), jnp.bfloat16),
            pltpu.VMEM((2, M_PER, TF), jnp.float32),
            pltpu.VMEM((NC, Q, CF), jnp.bfloat16),
            pltpu.VMEM((2 * Q, F_DIM), jnp.bfloat16),
            pltpu.VMEM((NC, Q, CF), jnp.bfloat16),
            pltpu.VMEM((NC, Q, CF), jnp.bfloat16),
            pltpu.VMEM((2 * Q, CF), jnp.float32),
            pltpu.SemaphoreType.DMA((2,)),
            pltpu.SemaphoreType.DMA,
            pltpu.SemaphoreType.DMA((NC,)),
            pltpu.SemaphoreType.DMA((NC,)),
            pltpu.SemaphoreType.DMA((NC,)),
            pltpu.SemaphoreType.DMA((NC,)),
        ],
        compiler_params=pltpu.CompilerParams(
            collective_id=0,
            vmem_limit_bytes=64 * 1024 * 1024,
        ),
    )(x, dy)


# device time: 92070 ns/iter; 4.5640x vs baseline; 2.0762x over previous
import jax
import jax.numpy as jnp
from jax import lax
from jax.experimental import pallas as pl
from jax.experimental.pallas import tpu as pltpu

M_PER = 2048
D_DIM = 2048
F_DIM = 8192
Q = 512
TF = 256
N_TF = F_DIM // TF
XC = 256
NC = 8
CF = F_DIM // NC
TPC = CF // TF
COMM = True


def kernel(x, dy):
    def body(x_hbm, dy_hbm, out_hbm,
             xstage, xall, dystage, e_buf, d_buf, r1_buf, r2_buf, out_stage,
             local_sems, out_sem, y_send_sems, y_recv_sems,
             z_send_sems, z_recv_sems):
        my_x = lax.axis_index("x")
        my_y = lax.axis_index("y")
        my_z = lax.axis_index("z")
        y_nbr = (my_x, 1 - my_y, my_z)
        z_nbr = (my_x, my_y, 1 - my_z)

        if COMM:
            barrier_sem = pltpu.get_barrier_semaphore()
            for nbr in (y_nbr, z_nbr):
                pl.semaphore_signal(barrier_sem, inc=1, device_id=nbr,
                                    device_id_type=pl.DeviceIdType.MESH)
            pl.semaphore_wait(barrier_sem, 2)

        def y_rdma(i):
            return pltpu.make_async_remote_copy(
                src_ref=e_buf.at[i], dst_ref=r1_buf.at[i],
                send_sem=y_send_sems.at[i], recv_sem=y_recv_sems.at[i],
                device_id=y_nbr, device_id_type=pl.DeviceIdType.MESH)

        def z_rdma(i):
            return pltpu.make_async_remote_copy(
                src_ref=r1_buf.at[i], dst_ref=r2_buf.at[i],
                send_sem=z_send_sems.at[i], recv_sem=z_recv_sems.at[i],
                device_id=z_nbr, device_id_type=pl.DeviceIdType.MESH)

        qe_col = (1 - my_y) * 1024 + my_z * Q
        qd_col = my_y * 1024
        xoffs = []
        for i, coloff in enumerate([qe_col, qd_col, qd_col + Q]):
            for c in range(Q // XC):
                xoffs.append((i * Q + c * XC, coloff + c * XC))
        cps = [None, None]
        for j, (dst_off, src_off) in enumerate(xoffs):
            slot = j % 2
            cps[slot] = pltpu.make_async_copy(
                x_hbm.at[:, pl.ds(src_off, XC)], xstage.at[slot],
                local_sems.at[slot])
            cps[slot].start()
            if j > 0:
                prev_dst = xoffs[j - 1][0]
                cps[(j - 1) % 2].wait()
                xall[:, prev_dst:prev_dst + XC] = (
                    xstage[(j - 1) % 2].astype(jnp.bfloat16))
        cps[(len(xoffs) - 1) % 2].wait()
        last_dst = xoffs[-1][0]
        xall[:, last_dst:last_dst + XC] = (
            xstage[(len(xoffs) - 1) % 2].astype(jnp.bfloat16))

        def dy_cp(t, slot):
            return pltpu.make_async_copy(
                dy_hbm.at[:, pl.ds(t * TF, TF)], dystage.at[slot],
                local_sems.at[slot])

        dy_cp(0, 0).start()

        def compute_step(t, carry):
            slot = t % 2

            @pl.when(t + 1 < N_TF)
            def _():
                dy_cp(t + 1, 1 - slot).start()

            dy_cp(t, slot).wait()
            dyt = dystage[slot].astype(jnp.bfloat16)
            res = lax.dot_general(
                xall[:, :], dyt,
                dimension_numbers=(((0,), (0,)), ((), ())),
                preferred_element_type=jnp.float32,
            )
            ci, cc = t // TPC, (t % TPC) * TF
            e_buf[ci, :, pl.ds(cc, TF)] = res[0:Q, :].astype(jnp.bfloat16)
            d_buf[:, pl.ds(t * TF, TF)] = res[Q:, :].astype(jnp.bfloat16)

            if COMM:
                @pl.when(t % TPC == TPC - 1)
                def _():
                    y_rdma(ci).start()

            return carry

        lax.fori_loop(0, N_TF, compute_step, 0)

        z_is_0 = (my_z == 0)

        def emit_out(i):
            d = d_buf[:, pl.ds(i * CF, CF)].astype(jnp.float32)
            r1 = r1_buf[i].astype(jnp.float32)
            r2 = r2_buf[i].astype(jnp.float32)
            out_stage[0:Q, :] = d[0:Q, :] + jnp.where(z_is_0, r1, r2)
            out_stage[Q:, :] = d[Q:, :] + jnp.where(z_is_0, r2, r1)
            pltpu.make_async_copy(
                out_stage, out_hbm.at[:, pl.ds(i * CF, CF)], out_sem).start()

        def out_wait(i):
            pltpu.make_async_copy(
                out_stage, out_hbm.at[:, pl.ds(i * CF, CF)], out_sem).wait()

        def pipe_step(i, carry):
            if COMM:
                y_rdma(i).wait_recv()
                z_rdma(i).start()

            @pl.when(i > 0)
            def _():
                if COMM:
                    z_rdma(i - 1).wait_recv()

                @pl.when(i > 1)
                def _():
                    out_wait(i - 2)

                emit_out(i - 1)

            return carry

        lax.fori_loop(0, NC, pipe_step, 0)
        if COMM:
            z_rdma(NC - 1).wait_recv()
        out_wait(NC - 2)
        emit_out(NC - 1)
        out_wait(NC - 1)

        if COMM:
            def drain_step(i, carry):
                y_rdma(i).wait_send()
                z_rdma(i).wait_send()
                return carry

            lax.fori_loop(0, NC, drain_step, 0)

    return pl.pallas_call(
        body,
        out_shape=jax.ShapeDtypeStruct((D_DIM // 2, F_DIM), jnp.float32),
        in_specs=[
            pl.BlockSpec(memory_space=pl.ANY),
            pl.BlockSpec(memory_space=pl.ANY),
        ],
        out_specs=pl.BlockSpec(memory_space=pl.ANY),
        scratch_shapes=[
            pltpu.VMEM((2, M_PER, XC), jnp.float32),
            pltpu.VMEM((M_PER, 3 * Q), jnp.bfloat16),
            pltpu.VMEM((2, M_PER, TF), jnp.float32),
            pltpu.VMEM((NC, Q, CF), jnp.bfloat16),
            pltpu.VMEM((2 * Q, F_DIM), jnp.bfloat16),
            pltpu.VMEM((NC, Q, CF), jnp.bfloat16),
            pltpu.VMEM((NC, Q, CF), jnp.bfloat16),
            pltpu.VMEM((2 * Q, CF), jnp.float32),
            pltpu.SemaphoreType.DMA((2,)),
            pltpu.SemaphoreType.DMA,
            pltpu.SemaphoreType.DMA((NC,)),
            pltpu.SemaphoreType.DMA((NC,)),
            pltpu.SemaphoreType.DMA((NC,)),
            pltpu.SemaphoreType.DMA((NC,)),
        ],
        compiler_params=pltpu.CompilerParams(
            collective_id=0 if COMM else None,
            vmem_limit_bytes=64 * 1024 * 1024,
        ),
    )(x, dy)
